# baseline (speedup 1.0000x reference)
import functools

import jax
import jax.numpy as jnp
from jax import lax
from jax.experimental import pallas as pl
from jax.experimental.pallas import tpu as pltpu
from jax.experimental.pallas import tpu_sc as plsc

SEQ = 256
C = 1024

_info = plsc.get_sparse_core_info()
_NC, _NS = _info.num_cores, _info.num_subcores
_NW = _NC * _NS
_ROWS = SEQ // _NW

_mesh = plsc.VectorSubcoreMesh(core_axis_name="c", subcore_axis_name="s")


@functools.partial(
    pl.kernel,
    mesh=_mesh,
    out_type=jax.ShapeDtypeStruct((SEQ, SEQ, 2 * C), jnp.float32),
)
def _encode(x_hbm, emb_hbm, out_hbm):
    wid = lax.axis_index("s") * _NC + lax.axis_index("c")
    base = wid * _ROWS
    pltpu.sync_copy(
        x_hbm.at[pl.ds(base, _ROWS)],
        out_hbm.at[pl.ds(base, _ROWS), :, pl.ds(0, C)],
    )
    for r in range(_ROWS):
        pltpu.sync_copy(
            emb_hbm.at[pl.ds(0, SEQ)],
            out_hbm.at[base + r, :, pl.ds(C, C)],
        )


def kernel(x, embedding):
    return _encode(x, embedding)


# speedup vs baseline: 46.6027x; 46.6027x over previous
import functools

import jax
import jax.numpy as jnp
from jax import lax
from jax.experimental import pallas as pl
from jax.experimental.pallas import tpu as pltpu
from jax.experimental.pallas import tpu_sc as plsc

SEQ = 256
C = 1024
CH = 32
JC = SEQ // CH

_info = plsc.get_sparse_core_info()
_NC, _NS = _info.num_cores, _info.num_subcores
_NW = _NC * _NS
_ROWS = SEQ // _NW

_mesh = plsc.VectorSubcoreMesh(core_axis_name="c", subcore_axis_name="s")


@functools.partial(
    pl.kernel,
    mesh=_mesh,
    out_type=jax.ShapeDtypeStruct((SEQ, SEQ, 2 * C), jnp.float32),
    scratch_types=[
        pltpu.VMEM((2, CH, C), jnp.float32),
        pltpu.VMEM((CH, C), jnp.float32),
        pltpu.SemaphoreType.DMA,
        pltpu.SemaphoreType.DMA,
        pltpu.SemaphoreType.DMA,
    ],
)
def _encode(x_hbm, emb_hbm, out_hbm, xbuf, ebuf, xin_sem, xout_sem, e_sem):
    wid = lax.axis_index("s") * _NC + lax.axis_index("c")
    base = wid * _ROWS

    def x_src(t):
        jc, ii = divmod(t, _ROWS)
        return x_hbm.at[base + ii, pl.ds(jc * CH, CH), :]

    def x_dst(t):
        jc, ii = divmod(t, _ROWS)
        return out_hbm.at[base + ii, pl.ds(jc * CH, CH), pl.ds(0, C)]

    def e_dst(t):
        jc, ii = divmod(t, _ROWS)
        return out_hbm.at[base + ii, pl.ds(jc * CH, CH), pl.ds(C, C)]

    T = JC * _ROWS
    xin = [None, None]
    xout = [None, None]
    eouts = []
    xin[0] = pltpu.async_copy(x_src(0), xbuf.at[0], xin_sem)
    for t in range(T):
        p = t & 1
        jc, ii = divmod(t, _ROWS)
        if ii == 0:
            for h in eouts:
                h.wait()
            eouts = []
            pltpu.sync_copy(emb_hbm.at[pl.ds(jc * CH, CH), :], ebuf)
        if t + 1 < T:
            if xout[1 - p] is not None:
                xout[1 - p].wait()
            xin[1 - p] = pltpu.async_copy(x_src(t + 1), xbuf.at[1 - p], xin_sem)
        xin[p].wait()
        xout[p] = pltpu.async_copy(xbuf.at[p], x_dst(t), xout_sem)
        eouts.append(pltpu.async_copy(ebuf, e_dst(t), e_sem))
    for h in eouts:
        h.wait()
    for h in xout:
        if h is not None:
            h.wait()


def kernel(x, embedding):
    return _encode(x, embedding)


# speedup vs baseline: 46.6750x; 1.0016x over previous
import functools

import jax
import jax.numpy as jnp
from jax import lax
from jax.experimental import pallas as pl
from jax.experimental.pallas import tpu as pltpu
from jax.experimental.pallas import tpu_sc as plsc

SEQ = 256
C = 1024
CH = 16
JC = SEQ // CH
NX = 4
LEAD = 2

_info = plsc.get_sparse_core_info()
_NC, _NS = _info.num_cores, _info.num_subcores
_NW = _NC * _NS
_ROWS = SEQ // _NW

_mesh = plsc.VectorSubcoreMesh(core_axis_name="c", subcore_axis_name="s")


@functools.partial(
    pl.kernel,
    mesh=_mesh,
    out_type=jax.ShapeDtypeStruct((SEQ, SEQ, 2 * C), jnp.float32),
    scratch_types=[
        pltpu.VMEM((NX, CH, C), jnp.float32),
        pltpu.VMEM((2, CH, C), jnp.float32),
        pltpu.SemaphoreType.DMA,
        pltpu.SemaphoreType.DMA,
        pltpu.SemaphoreType.DMA,
        pltpu.SemaphoreType.DMA,
    ],
)
def _encode(x_hbm, emb_hbm, out_hbm, xbuf, ebuf, xin_sem, xout_sem,
            ein_sem, eout_sem):
    wid = lax.axis_index("s") * _NC + lax.axis_index("c")
    base = wid * _ROWS

    def x_src(t):
        jc, ii = divmod(t, _ROWS)
        return x_hbm.at[base + ii, pl.ds(jc * CH, CH), :]

    def x_dst(t):
        jc, ii = divmod(t, _ROWS)
        return out_hbm.at[base + ii, pl.ds(jc * CH, CH), pl.ds(0, C)]

    def e_dst(t):
        jc, ii = divmod(t, _ROWS)
        return out_hbm.at[base + ii, pl.ds(jc * CH, CH), pl.ds(C, C)]

    T = JC * _ROWS
    xin = [None] * NX
    xout = [None] * NX
    ein = [None, None]
    eouts = [[], []]

    ein[0] = pltpu.async_copy(emb_hbm.at[pl.ds(0, CH), :], ebuf.at[0], ein_sem)
    for t in range(LEAD + 1):
        xin[t % NX] = pltpu.async_copy(x_src(t), xbuf.at[t % NX], xin_sem)

    for t in range(T):
        p = t % NX
        jc, ii = divmod(t, _ROWS)
        ep = jc & 1
        if ii == 0:
            ein[ep].wait()
            if jc + 1 < JC:
                for h in eouts[1 - ep]:
                    h.wait()
                eouts[1 - ep] = []
                ein[1 - ep] = pltpu.async_copy(
                    emb_hbm.at[pl.ds((jc + 1) * CH, CH), :], ebuf.at[1 - ep],
                    ein_sem)
        nt = t + LEAD + 1
        if nt < T:
            r = nt % NX
            if xout[r] is not None:
                xout[r].wait()
            xin[r] = pltpu.async_copy(x_src(nt), xbuf.at[r], xin_sem)
        xin[p].wait()
        xout[p] = pltpu.async_copy(xbuf.at[p], x_dst(t), xout_sem)
        eouts[ep].append(pltpu.async_copy(ebuf.at[ep], e_dst(t), eout_sem))

    for hs in eouts:
        for h in hs:
            h.wait()
    for h in xout:
        if h is not None:
            h.wait()


def kernel(x, embedding):
    return _encode(x, embedding)
